# Initial kernel scaffold; baseline (speedup 1.0000x reference)
import jax
import jax.numpy as jnp
from jax.experimental import pallas as pl


def kernel(x, cb0, cb1, cb2, cb3):
    raise NotImplementedError("write your pallas kernel here")


# speedup vs baseline: 1.6429x; 1.6429x over previous
import functools

import jax
import jax.numpy as jnp
from jax.experimental import pallas as pl
from jax.experimental.pallas import tpu as pltpu

_NUM_LEVELS = 4
_K = 1024
_D = 256
_N = 8192
_B = 512


def _vq_kernel(x_ref, cbsq_ref,
               cb0, cb1, cb2, cb3,
               h0, h1, h2, h3,
               m0, m1, m2, m3,
               l0, l1, l2, l3,
               idx_ref, r_ref, e_ref, z_ref, cnt_ref,
               hist_ref):
    i = pl.program_id(0)
    nb = pl.num_programs(0)

    @pl.when(i == 0)
    def _init():
        hist_ref[...] = jnp.zeros_like(hist_ref)

    cbs = (cb0, cb1, cb2, cb3)
    his = (h0, h1, h2, h3)
    mis = (m0, m1, m2, m3)
    los = (l0, l1, l2, l3)

    x0 = x_ref[...]
    xcur = x0
    ids = jax.lax.broadcasted_iota(jnp.int32, (_B, _K), 1)
    qsum = None
    for l in range(_NUM_LEVELS):
        cb = cbs[l][...]
        xc = jax.lax.dot_general(
            xcur, cb, (((1,), (1,)), ((), ())),
            preferred_element_type=jnp.float32)
        xx = jnp.sum(xcur * xcur, axis=1, keepdims=True)
        d2 = (xx - 2.0 * xc) + cbsq_ref[l, :][None, :]
        dist = jnp.sqrt(jnp.maximum(d2, 0.0))
        mind = jnp.min(dist, axis=1, keepdims=True)
        idx = jnp.min(jnp.where(dist == mind, ids, _K), axis=1)
        onehot = (ids == idx[:, None]).astype(jnp.bfloat16)
        qhi = jax.lax.dot_general(
            onehot, his[l][...], (((1,), (0,)), ((), ())),
            preferred_element_type=jnp.float32)
        qmi = jax.lax.dot_general(
            onehot, mis[l][...], (((1,), (0,)), ((), ())),
            preferred_element_type=jnp.float32)
        qlo = jax.lax.dot_general(
            onehot, los[l][...], (((1,), (0,)), ((), ())),
            preferred_element_type=jnp.float32)
        q = (qhi + qmi) + qlo
        idx_ref[:, l:l + 1] = idx[:, None]
        r_ref[:, l, :] = xcur
        e_ref[:, l, :] = q
        colsum = jnp.sum(onehot.astype(jnp.float32), axis=0, keepdims=True)
        hist_ref[l:l + 1, :] += colsum
        qsum = q if qsum is None else qsum + q
        xcur = xcur - q
    z_ref[...] = qsum

    @pl.when(i == nb - 1)
    def _finish():
        used = hist_ref[0:_NUM_LEVELS, :]
        cnt_ref[...] = jnp.sum((used < 1.0).astype(jnp.int32),
                               axis=(0, 1), keepdims=True)


@functools.partial(jax.jit, static_argnames=())
def kernel(x, cb0, cb1, cb2, cb3):
    cbs = [cb0, cb1, cb2, cb3]
    cbsq = jnp.stack([jnp.sum(cb * cb, axis=1) for cb in cbs], axis=0)
    his, mis, los = [], [], []
    for cb in cbs:
        hi = jax.lax.optimization_barrier(cb.astype(jnp.bfloat16))
        hi32 = jax.lax.optimization_barrier(hi.astype(jnp.float32))
        mid = jax.lax.optimization_barrier((cb - hi32).astype(jnp.bfloat16))
        mid32 = jax.lax.optimization_barrier(mid.astype(jnp.float32))
        lo = (cb - hi32 - mid32).astype(jnp.bfloat16)
        his.append(hi)
        mis.append(mid)
        los.append(lo)

    nb = _N // _B
    full = lambda i: (0, 0)
    in_specs = [
            pl.BlockSpec((_B, _D), lambda i: (i, 0)),
            pl.BlockSpec((_NUM_LEVELS, _K), full),
    ] + [pl.BlockSpec((_K, _D), full)] * 16
    out_specs = [
        pl.BlockSpec((_B, _NUM_LEVELS), lambda i: (i, 0)),
        pl.BlockSpec((_B, _NUM_LEVELS, _D), lambda i: (i, 0, 0)),
        pl.BlockSpec((_B, _NUM_LEVELS, _D), lambda i: (i, 0, 0)),
        pl.BlockSpec((_B, _D), lambda i: (i, 0)),
        pl.BlockSpec((1, 1), full),
    ]
    out_shapes = [
        jax.ShapeDtypeStruct((_N, _NUM_LEVELS), jnp.int32),
        jax.ShapeDtypeStruct((_N, _NUM_LEVELS, _D), jnp.float32),
        jax.ShapeDtypeStruct((_N, _NUM_LEVELS, _D), jnp.float32),
        jax.ShapeDtypeStruct((_N, _D), jnp.float32),
        jax.ShapeDtypeStruct((1, 1), jnp.int32),
    ]
    idx, r_s, e_s, z_hat, cnt = pl.pallas_call(
        _vq_kernel,
        grid=(nb,),
        in_specs=in_specs,
        out_specs=out_specs,
        out_shape=out_shapes,
        scratch_shapes=[pltpu.VMEM((8, _K), jnp.float32)],
        compiler_params=pltpu.CompilerParams(
            dimension_semantics=("arbitrary",),
        ),
    )(x, cbsq, *cbs, *his, *mis, *los)
    return (idx.astype(jnp.int64), r_s, e_s, z_hat,
            jnp.reshape(cnt, ()))
